# Initial kernel scaffold; baseline (speedup 1.0000x reference)
import jax
import jax.numpy as jnp
from jax.experimental import pallas as pl


def kernel(output, mask, ind, target):
    raise NotImplementedError("write your pallas kernel here")


# speedup vs baseline: 1.1225x; 1.1225x over previous
import jax
import jax.numpy as jnp
from jax import lax
from jax.experimental import pallas as pl
from jax.experimental.pallas import tpu as pltpu
from jax.experimental.pallas import tpu_sc as plsc

NC, NS, L = 2, 16, 16
NW = NC * NS
B, C, H, W, M = 64, 16, 152, 152, 512
HW = H * W
B_PER_W = B // NW
NPLANE = B_PER_W * C
MV = M // L


def _body(feat_hbm, ind_hbm, maskf_hbm, tgt_hbm, loss_hbm, nv_hbm,
          ind_v, mk_v, tg_v, plane0_v, plane1_v, stage_v, sem0, sem1):
    wid = lax.axis_index("s") * NC + lax.axis_index("c")
    b0 = wid * B_PER_W
    sems = (sem0, sem1)
    planes = (plane0_v, plane1_v)

    def plane_copy(p, slot):
        b = b0 + p // C
        c = p - (p // C) * C
        return pltpu.make_async_copy(
            feat_hbm.at[b * C + c], planes[slot], sems[slot])

    plane_copy(0, 0).start()
    plane_copy(1, 1).start()
    for i in range(B_PER_W):
        b = b0 + i
        pltpu.sync_copy(ind_hbm.at[b], ind_v.at[i])
        pltpu.sync_copy(maskf_hbm.at[b], mk_v.at[i])
        pltpu.sync_copy(tgt_hbm.at[b], tg_v.at[i])

    def pair_step(q, carry):
        acc = carry
        for slot in range(2):
            p = 2 * q + slot
            plane_copy(p, slot).wait()

            @pl.when(p + 2 < NPLANE)
            def _():
                plane_copy(p + 2, slot).start()

            i = p // C
            c = p - i * C
            plane = planes[slot]

            def accum(g, a):
                idx = ind_v[i, pl.ds(g * L, L)]
                pred = plsc.load_gather(plane, [idx])
                tg = tg_v[i, c, pl.ds(g * L, L)]
                mk = mk_v[i, pl.ds(g * L, L)]
                return a + mk * jnp.abs(pred - tg)

            acc = lax.fori_loop(0, MV, accum, acc)
        return acc

    acc = lax.fori_loop(0, NPLANE // 2, pair_step,
                        jnp.zeros((L,), jnp.float32))

    def msum(t, a):
        return a + mk_v[t // MV, pl.ds((t - (t // MV) * MV) * L, L)]

    nv = lax.fori_loop(0, B_PER_W * MV, msum, jnp.zeros((L,), jnp.float32))

    stage_v[...] = acc
    pltpu.sync_copy(stage_v, loss_hbm.at[wid])
    stage_v[...] = nv
    pltpu.sync_copy(stage_v, nv_hbm.at[wid])


@jax.jit
def kernel(output, mask, ind, target):
    feat = output.reshape(B * C, HW)
    maskf = mask.astype(jnp.float32)
    tgt = jnp.swapaxes(target, 1, 2)

    mesh = plsc.VectorSubcoreMesh(core_axis_name="c", subcore_axis_name="s")
    loss_parts, nv_parts = pl.kernel(
        _body,
        out_type=[
            jax.ShapeDtypeStruct((NW, L), jnp.float32),
            jax.ShapeDtypeStruct((NW, L), jnp.float32),
        ],
        mesh=mesh,
        compiler_params=pltpu.CompilerParams(needs_layout_passes=False),
        scratch_types=[
            pltpu.VMEM((B_PER_W, M), jnp.int32),
            pltpu.VMEM((B_PER_W, M), jnp.float32),
            pltpu.VMEM((B_PER_W, C, M), jnp.float32),
            pltpu.VMEM((HW,), jnp.float32),
            pltpu.VMEM((HW,), jnp.float32),
            pltpu.VMEM((L,), jnp.float32),
            pltpu.SemaphoreType.DMA,
            pltpu.SemaphoreType.DMA,
        ],
    )(feat, ind, maskf, tgt)

    s = loss_parts.sum()
    n_valid = nv_parts.sum()
    return jnp.where(n_valid > 0, s / jnp.maximum(n_valid, 1.0),
                     jnp.float32(0.0))


# speedup vs baseline: 2.1727x; 1.9356x over previous
import jax
import jax.numpy as jnp
from jax import lax
from jax.experimental import pallas as pl
from jax.experimental.pallas import tpu as pltpu
from jax.experimental.pallas import tpu_sc as plsc

NC, NS, L = 2, 16, 16
NW = NC * NS
B, C, H, W, M = 64, 16, 152, 152, 512
HW = H * W
B_PER_W = B // NW
NPLANE = B_PER_W * C
MV = M // L


def _body(feat_hbm, ind_hbm, maskf_hbm, tgt_hbm, loss_hbm, nv_hbm,
          row_v, col_v, mk_v, tg_v, plane0_v, plane1_v, stage_v, sem0, sem1):
    wid = lax.axis_index("s") * NC + lax.axis_index("c")
    b0 = wid * B_PER_W
    sems = (sem0, sem1)
    planes = (plane0_v, plane1_v)

    def plane_copy(p, slot):
        b = b0 + p // C
        c = p - (p // C) * C
        return pltpu.make_async_copy(
            feat_hbm.at[b, c], planes[slot], sems[slot])

    plane_copy(0, 0).start()
    plane_copy(1, 1).start()
    for i in range(B_PER_W):
        b = b0 + i
        pltpu.sync_copy(ind_hbm.at[b], row_v.at[i])
        pltpu.sync_copy(maskf_hbm.at[b], mk_v.at[i])
        pltpu.sync_copy(tgt_hbm.at[b], tg_v.at[i])

        def split(t, carry):
            v = row_v[i, pl.ds(t * L, L)]
            r = lax.div(v, W)
            col_v[i, pl.ds(t * L, L)] = v - r * W
            row_v[i, pl.ds(t * L, L)] = r
            return carry

        lax.fori_loop(0, MV, split, 0)

    def pair_step(q, carry):
        acc = carry
        for slot in range(2):
            p = 2 * q + slot
            plane_copy(p, slot).wait()

            @pl.when(p + 2 < NPLANE)
            def _():
                plane_copy(p + 2, slot).start()

            i = p // C
            c = p - i * C
            plane = planes[slot]

            def accum(g, a):
                r = row_v[i, pl.ds(g * L, L)]
                cc = col_v[i, pl.ds(g * L, L)]
                pred = plsc.load_gather(plane, [r, cc])
                tg = tg_v[i, c, pl.ds(g * L, L)]
                mk = mk_v[i, pl.ds(g * L, L)]
                return a + mk * jnp.abs(pred - tg)

            acc = lax.fori_loop(0, MV, accum, acc)
        return acc

    acc = lax.fori_loop(0, NPLANE // 2, pair_step,
                        jnp.zeros((L,), jnp.float32))

    def msum(t, a):
        return a + mk_v[t // MV, pl.ds((t - (t // MV) * MV) * L, L)]

    nv = lax.fori_loop(0, B_PER_W * MV, msum, jnp.zeros((L,), jnp.float32))

    stage_v[...] = acc
    pltpu.sync_copy(stage_v, loss_hbm.at[wid])
    stage_v[...] = nv
    pltpu.sync_copy(stage_v, nv_hbm.at[wid])


@jax.jit
def kernel(output, mask, ind, target):
    maskf = mask.astype(jnp.float32)
    tgt = jnp.swapaxes(target, 1, 2)

    mesh = plsc.VectorSubcoreMesh(core_axis_name="c", subcore_axis_name="s")
    loss_parts, nv_parts = pl.kernel(
        _body,
        out_type=[
            jax.ShapeDtypeStruct((NW, L), jnp.float32),
            jax.ShapeDtypeStruct((NW, L), jnp.float32),
        ],
        mesh=mesh,
        compiler_params=pltpu.CompilerParams(needs_layout_passes=False),
        scratch_types=[
            pltpu.VMEM((B_PER_W, M), jnp.int32),
            pltpu.VMEM((B_PER_W, M), jnp.int32),
            pltpu.VMEM((B_PER_W, M), jnp.float32),
            pltpu.VMEM((B_PER_W, C, M), jnp.float32),
            pltpu.VMEM((H, W), jnp.float32),
            pltpu.VMEM((H, W), jnp.float32),
            pltpu.VMEM((L,), jnp.float32),
            pltpu.SemaphoreType.DMA,
            pltpu.SemaphoreType.DMA,
        ],
    )(output, ind, maskf, tgt)

    s = loss_parts.sum()
    n_valid = nv_parts.sum()
    return jnp.where(n_valid > 0, s / jnp.maximum(n_valid, 1.0),
                     jnp.float32(0.0))


# speedup vs baseline: 2.1780x; 1.0024x over previous
import jax
import jax.numpy as jnp
from jax import lax
from jax.experimental import pallas as pl
from jax.experimental.pallas import tpu as pltpu
from jax.experimental.pallas import tpu_sc as plsc

NC, NS, L = 2, 16, 16
NW = NC * NS
B, C, H, W, M = 64, 16, 152, 152, 512
B_PER_W = B // NW
NPLANE = B_PER_W * C
MV = M // L


def _body(feat_hbm, ind_hbm, mask_hbm, tgt_hbm, out_hbm,
          row_v, col_v, mki_v, mk_v, tg_v, plane0_v, plane1_v, stage_v,
          sem0, sem1):
    wid = lax.axis_index("s") * NC + lax.axis_index("c")
    b0 = wid * B_PER_W
    sems = (sem0, sem1)
    planes = (plane0_v, plane1_v)

    def plane_copy(p, slot):
        b = b0 + p // C
        c = p - (p // C) * C
        return pltpu.make_async_copy(
            feat_hbm.at[b, c], planes[slot], sems[slot])

    plane_copy(0, 0).start()
    plane_copy(1, 1).start()
    for i in range(B_PER_W):
        b = b0 + i
        pltpu.sync_copy(ind_hbm.at[b], row_v.at[i])
        pltpu.sync_copy(mask_hbm.at[b], mki_v.at[i])
        pltpu.sync_copy(tgt_hbm.at[b], tg_v.at[i])

        @plsc.parallel_loop(0, MV, unroll=4)
        def _split(t):
            v = row_v[i, pl.ds(t * L, L)]
            r = lax.div(v, W)
            col_v[i, pl.ds(t * L, L)] = v - r * W
            row_v[i, pl.ds(t * L, L)] = r
            mk_v[i, pl.ds(t * L, L)] = mki_v[i, pl.ds(t * L, L)].astype(
                jnp.float32)

    def pair_step(q, carry):
        acc0 = carry
        for slot in range(2):
            p = 2 * q + slot
            plane_copy(p, slot).wait()

            @pl.when(p + 2 < NPLANE)
            def _():
                plane_copy(p + 2, slot).start()

            i = p // C
            c = p - i * C
            plane = planes[slot]

            @plsc.parallel_loop(0, MV, unroll=8, carry=acc0)
            def accum(g, a):
                r = row_v[i, pl.ds(g * L, L)]
                cc = col_v[i, pl.ds(g * L, L)]
                pred = plsc.load_gather(plane, [r, cc])
                tg = tg_v[i, c, pl.ds(g * L, L)]
                mk = mk_v[i, pl.ds(g * L, L)]
                return a + mk * jnp.abs(pred - tg)

            acc0 = accum
        return acc0

    acc = lax.fori_loop(0, NPLANE // 2, pair_step,
                        jnp.zeros((L,), jnp.float32))

    @plsc.parallel_loop(0, B_PER_W * MV, unroll=4,
                        carry=jnp.zeros((L,), jnp.float32))
    def nv(t, a):
        return a + mk_v[t // MV, pl.ds((t - (t // MV) * MV) * L, L)]

    stage_v[...] = acc
    pltpu.sync_copy(stage_v, out_hbm.at[0, wid])
    stage_v[...] = nv
    pltpu.sync_copy(stage_v, out_hbm.at[1, wid])


@jax.jit
def kernel(output, mask, ind, target):
    tgt = jnp.swapaxes(target, 1, 2)

    mesh = plsc.VectorSubcoreMesh(core_axis_name="c", subcore_axis_name="s")
    parts = pl.kernel(
        _body,
        out_type=jax.ShapeDtypeStruct((2, NW, L), jnp.float32),
        mesh=mesh,
        compiler_params=pltpu.CompilerParams(needs_layout_passes=False),
        scratch_types=[
            pltpu.VMEM((B_PER_W, M), jnp.int32),
            pltpu.VMEM((B_PER_W, M), jnp.int32),
            pltpu.VMEM((B_PER_W, M), jnp.int32),
            pltpu.VMEM((B_PER_W, M), jnp.float32),
            pltpu.VMEM((B_PER_W, C, M), jnp.float32),
            pltpu.VMEM((H, W), jnp.float32),
            pltpu.VMEM((H, W), jnp.float32),
            pltpu.VMEM((L,), jnp.float32),
            pltpu.SemaphoreType.DMA,
            pltpu.SemaphoreType.DMA,
        ],
    )(output, ind, mask, tgt)

    sums = parts.sum(axis=(1, 2))
    s, n_valid = sums[0], sums[1]
    return jnp.where(n_valid > 0, s / jnp.maximum(n_valid, 1.0),
                     jnp.float32(0.0))
